# Initial kernel scaffold; baseline (speedup 1.0000x reference)
import jax
import jax.numpy as jnp
from jax.experimental import pallas as pl


def kernel(x, embed_table, time_decay, Wr, Wk, Wv, Wo, mem_keys, mem_values, Wc, Wd, W_out, b_out):
    raise NotImplementedError("write your pallas kernel here")


# speedup vs baseline: 44.5613x; 44.5613x over previous
import functools

import jax
import jax.numpy as jnp
from jax import lax
from jax.experimental import pallas as pl
from jax.experimental.pallas import tpu as pltpu
from jax.experimental.pallas import tpu_sc as plsc

_R = 512


def _embed_gather(x_flat, table):
    n_tok = x_flat.shape[0]
    d = table.shape[1]
    info = plsc.get_sparse_core_info()
    nc, ns = info.num_cores, info.num_subcores
    nw = nc * ns
    b_per_w = n_tok // nw

    mesh = plsc.VectorSubcoreMesh(core_axis_name="c", subcore_axis_name="s")

    @functools.partial(
        pl.kernel,
        mesh=mesh,
        out_type=jax.ShapeDtypeStruct((n_tok, d), jnp.float32),
        scratch_types=[
            pltpu.VMEM((b_per_w,), jnp.int32),
            pltpu.VMEM((b_per_w, d), jnp.float32),
            pltpu.SemaphoreType.DMA,
        ],
    )
    def gather_kernel(idx_hbm, table_hbm, out_hbm, idx_v, rows_v, sem):
        wid = lax.axis_index("s") * nc + lax.axis_index("c")
        base = wid * b_per_w
        pltpu.sync_copy(idx_hbm.at[pl.ds(base, b_per_w)], idx_v)
        pltpu.async_copy(table_hbm.at[idx_v], rows_v, sem).wait()
        pltpu.sync_copy(rows_v, out_hbm.at[pl.ds(base, b_per_w)])

    return gather_kernel(x_flat, table)


def _mega_body(cpb, cap, h_ref, td_ref, wr_ref, wk_ref, wv_ref, wo_ref,
               mk_ref, mv_ref, wc_ref, wd_ref, wout_ref, bout_ref,
               out_ref, cn_ref, cd_ref):
    i = pl.program_id(0)
    rr = h_ref.shape[0]

    @pl.when(i % cpb == 0)
    def _():
        cn_ref[...] = jnp.zeros_like(cn_ref)
        cd_ref[...] = jnp.zeros_like(cd_ref)

    h = h_ref[...]
    e = jnp.exp(td_ref[...])

    r = jax.nn.sigmoid(jnp.dot(h, wr_ref[...], preferred_element_type=jnp.float32))
    k = jnp.dot(h, wk_ref[...], preferred_element_type=jnp.float32)
    v = jnp.dot(h, wv_ref[...], preferred_element_type=jnp.float32)
    ek = jnp.exp(jnp.clip(k, -30.0, 30.0))

    num = ek * v
    den = ek
    s = 1
    while s < rr:
        dk = jnp.exp(jnp.float32(-s) * e)
        zpad = jnp.zeros((s, num.shape[1]), jnp.float32)
        num = num + dk * jnp.concatenate([zpad, num[:-s, :]], axis=0)
        den = den + dk * jnp.concatenate([zpad, den[:-s, :]], axis=0)
        s *= 2

    tpos = lax.broadcasted_iota(jnp.int32, (rr, 1), 0).astype(jnp.float32) + 1.0
    tpow = jnp.exp(-tpos * e)
    num = num + tpow * cn_ref[...]
    den = den + tpow * cd_ref[...]
    cn_ref[...] = num[rr - 1:rr, :]
    cd_ref[...] = den[rr - 1:rr, :]
    wkv = num / (den + 1e-6)

    h2 = h + jnp.dot(r * wkv, wo_ref[...], preferred_element_type=jnp.float32)

    q = jnp.dot(h2, wc_ref[...], preferred_element_type=jnp.float32)
    c_dim = q.shape[1]
    scores = lax.dot_general(q, mk_ref[...], (((1,), (1,)), ((), ())),
                             preferred_element_type=jnp.float32)
    scores = scores * jnp.float32(1.0 / (c_dim ** 0.5))

    col = lax.broadcasted_iota(jnp.int32, (rr, cap), 1)
    m1 = jnp.max(scores, axis=1, keepdims=True)
    i1 = jnp.min(jnp.where(scores == m1, col, cap), axis=1, keepdims=True)
    masked = jnp.where(col == i1, jnp.float32(-jnp.inf), scores)
    m2 = jnp.max(masked, axis=1, keepdims=True)
    i2 = jnp.min(jnp.where(masked == m2, col, cap), axis=1, keepdims=True)
    e2 = jnp.exp(m2 - m1)
    w1 = 1.0 / (1.0 + e2)
    w2 = e2 / (1.0 + e2)
    wsel = jnp.where(col == i1, w1, 0.0) + jnp.where(col == i2, w2, 0.0)

    read = jnp.dot(wsel, mv_ref[...], preferred_element_type=jnp.float32)
    h3 = h2 + jnp.dot(read, wd_ref[...], preferred_element_type=jnp.float32)
    out_ref[...] = (jnp.dot(h3, wout_ref[...], preferred_element_type=jnp.float32)
                    + bout_ref[...])


def _tc_forward(h, td, wr, wk, wv, wo, mem_keys, mem_values, wc, wd,
                w_out, b_out, chunks_per_batch):
    n_tok, d = h.shape
    cap, c = mem_keys.shape
    vocab = w_out.shape[1]
    n_chunks = n_tok // _R

    fixed = lambda i: (0, 0)
    return pl.pallas_call(
        functools.partial(_mega_body, chunks_per_batch, cap),
        grid=(n_chunks,),
        in_specs=[
            pl.BlockSpec((_R, d), lambda i: (i, 0)),
            pl.BlockSpec((1, d), fixed),
            pl.BlockSpec((d, d), fixed),
            pl.BlockSpec((d, d), fixed),
            pl.BlockSpec((d, d), fixed),
            pl.BlockSpec((d, d), fixed),
            pl.BlockSpec((cap, c), fixed),
            pl.BlockSpec((cap, c), fixed),
            pl.BlockSpec((d, c), fixed),
            pl.BlockSpec((c, d), fixed),
            pl.BlockSpec((d, vocab), fixed),
            pl.BlockSpec((1, vocab), fixed),
        ],
        out_specs=pl.BlockSpec((_R, vocab), lambda i: (i, 0)),
        out_shape=jax.ShapeDtypeStruct((n_tok, vocab), jnp.float32),
        scratch_shapes=[
            pltpu.VMEM((1, d), jnp.float32),
            pltpu.VMEM((1, d), jnp.float32),
        ],
        compiler_params=pltpu.CompilerParams(
            dimension_semantics=("arbitrary",),
        ),
    )(h, td, wr, wk, wv, wo, mem_keys, mem_values, wc, wd, w_out, b_out)


def kernel(x, embed_table, time_decay, Wr, Wk, Wv, Wo, mem_keys, mem_values,
           Wc, Wd, W_out, b_out):
    b, s = x.shape
    vocab = W_out.shape[1]
    x_flat = x.reshape(-1).astype(jnp.int32)
    h = _embed_gather(x_flat, embed_table)
    out = _tc_forward(h, time_decay.reshape(1, -1), Wr, Wk, Wv, Wo,
                      mem_keys, mem_values, Wc, Wd, W_out,
                      b_out.reshape(1, -1), chunks_per_batch=s // _R)
    return out.reshape(b, s, vocab)
